# baseline (speedup 1.0000x reference)
import functools

import jax
import jax.numpy as jnp
from jax import lax
from jax.experimental import pallas as pl
from jax.experimental.pallas import tpu as pltpu
from jax.experimental.pallas import tpu_sc as plsc

_N_EXPERTS = 16
_TOPK = 2
_BT = 1024

_NC = 2
_NS = 16
_NW = _NC * _NS
_LANES = 16


def _matmul_body(x_ref, w_ref, out_ref):
    out_ref[...] = lax.dot_general(
        w_ref[...], x_ref[...], (((1,), (1,)), ((), ())),
        preferred_element_type=jnp.float32)


def _tc_logits_t(x, W):
    tokens, dim = x.shape
    n_experts = W.shape[0]
    return pl.pallas_call(
        _matmul_body,
        grid=(tokens // _BT,),
        in_specs=[
            pl.BlockSpec((_BT, dim), lambda i: (i, 0)),
            pl.BlockSpec((n_experts, dim), lambda i: (0, 0)),
        ],
        out_specs=pl.BlockSpec((n_experts, _BT), lambda i: (0, i)),
        out_shape=jax.ShapeDtypeStruct((n_experts, tokens), jnp.float32),
    )(x, W)


def _sc_router_body(logits_hbm, bias_hbm, w_out_hbm, idx_out_hbm,
                    logit_v, bias_v, w_v, i_v):
    n_tok = logit_v.shape[1]
    wid = lax.axis_index("s") * _NC + lax.axis_index("c")
    tok0 = wid * n_tok
    pltpu.sync_copy(logits_hbm.at[:, pl.ds(tok0, n_tok)], logit_v)
    pltpu.sync_copy(bias_hbm, bias_v)

    lane = lax.iota(jnp.int32, 16)
    zero = jnp.zeros((16,), jnp.int32)
    one = jnp.full((16,), 1, jnp.int32)
    e_vecs = [jnp.full((16,), e, jnp.int32) for e in range(_N_EXPERTS)]
    bias_b = [plsc.load_gather(bias_v, [e_vecs[e]]) for e in range(_N_EXPERTS)]

    def _comb(A, B):
        (a1m, a1i, a2m, a2i), (b1m, b1i, b2m, b2i) = A, B
        c1 = b1m > a1m
        c2 = b1m > a2m
        c3 = b2m > a1m
        n1m = jnp.where(c1, b1m, a1m)
        n1i = jnp.where(c1, b1i, a1i)
        n2m = jnp.where(c1, jnp.where(c3, b2m, a1m), jnp.where(c2, b1m, a2m))
        n2i = jnp.where(c1, jnp.where(c3, b2i, a1i), jnp.where(c2, b1i, a2i))
        return (n1m, n1i, n2m, n2i)

    def group(t, carry):
        base = t * _LANES
        tok = base + lane
        s_sel = []
        for e in range(_N_EXPERTS):
            z = logit_v[e, pl.ds(base, _LANES)]
            s_sel.append(1.0 / (1.0 + jnp.exp(-z)) + bias_b[e])
        nodes = []
        for e in range(0, _N_EXPERTS, 2):
            c = s_sel[e + 1] > s_sel[e]
            nodes.append((
                jnp.where(c, s_sel[e + 1], s_sel[e]),
                jnp.where(c, e_vecs[e + 1], e_vecs[e]),
                jnp.where(c, s_sel[e], s_sel[e + 1]),
                jnp.where(c, e_vecs[e], e_vecs[e + 1]),
            ))
        while len(nodes) > 1:
            nodes = [_comb(nodes[i], nodes[i + 1])
                     for i in range(0, len(nodes), 2)]
        m1, i1, m2, i2 = nodes[0]
        w1 = m1 - plsc.load_gather(bias_v, [i1])
        w2 = m2 - plsc.load_gather(bias_v, [i2])
        denom = jnp.maximum(w1 + w2, 1e-12)
        scale = 1.0 / denom
        pos = tok * _TOPK
        plsc.store_scatter(w_v, [pos], w1 * scale)
        plsc.store_scatter(w_v, [pos + 1], w2 * scale)
        plsc.store_scatter(i_v, [pos], i1)
        plsc.store_scatter(i_v, [pos + 1], i2)
        return carry

    lax.fori_loop(0, n_tok // _LANES, group, 0, unroll=2)

    out_base = wid * n_tok * _TOPK
    pltpu.sync_copy(w_v, w_out_hbm.at[pl.ds(out_base, n_tok * _TOPK)])
    pltpu.sync_copy(i_v, idx_out_hbm.at[pl.ds(out_base, n_tok * _TOPK)])


def _sc_router(logits_t, bias):
    tokens = logits_t.shape[1]
    n_tok = tokens // _NW
    mesh = plsc.VectorSubcoreMesh(core_axis_name="c", subcore_axis_name="s")
    run = pl.kernel(
        _sc_router_body,
        out_type=[
            jax.ShapeDtypeStruct((tokens * _TOPK,), jnp.float32),
            jax.ShapeDtypeStruct((tokens * _TOPK,), jnp.int32),
        ],
        mesh=mesh,
        scratch_types=[
            pltpu.VMEM((_N_EXPERTS, n_tok), jnp.float32),
            pltpu.VMEM((_N_EXPERTS,), jnp.float32),
            pltpu.VMEM((n_tok * _TOPK,), jnp.float32),
            pltpu.VMEM((n_tok * _TOPK,), jnp.int32),
        ],
        compiler_params=pltpu.CompilerParams(needs_layout_passes=False),
    )
    return run(logits_t, bias)


@jax.jit
def kernel(x, W, bias):
    tokens = x.shape[0]
    logits_t = _tc_logits_t(x, W)
    w_flat, i_flat = _sc_router(logits_t, bias)
    return w_flat.reshape(tokens, _TOPK), i_flat.reshape(tokens, _TOPK)


# speedup vs baseline: 1.1259x; 1.1259x over previous
import functools

import jax
import jax.numpy as jnp
from jax import lax
from jax.experimental import pallas as pl
from jax.experimental.pallas import tpu as pltpu
from jax.experimental.pallas import tpu_sc as plsc

_N_EXPERTS = 16
_TOPK = 2
_BT = 1024

_NC = 2
_NS = 16
_NW = _NC * _NS
_LANES = 16


def _matmul_body(x_ref, w_ref, out_ref):
    out_ref[...] = lax.dot_general(
        w_ref[...], x_ref[...], (((1,), (1,)), ((), ())),
        preferred_element_type=jnp.float32)


def _tc_logits_t(x, W):
    tokens, dim = x.shape
    n_experts = W.shape[0]
    return pl.pallas_call(
        _matmul_body,
        grid=(tokens // _BT,),
        in_specs=[
            pl.BlockSpec((_BT, dim), lambda i: (i, 0)),
            pl.BlockSpec((n_experts, dim), lambda i: (0, 0)),
        ],
        out_specs=pl.BlockSpec((n_experts, _BT), lambda i: (0, i)),
        out_shape=jax.ShapeDtypeStruct((n_experts, tokens), jnp.float32),
    )(x, W)


def _sc_router_body(logits_hbm, bias_hbm, w_out_hbm, idx_out_hbm,
                    logit_v, bias_v, w_v, i_v):
    n_tok = logit_v.shape[1]
    wid = lax.axis_index("s") * _NC + lax.axis_index("c")
    tok0 = wid * n_tok
    pltpu.sync_copy(logits_hbm.at[:, pl.ds(tok0, n_tok)], logit_v)
    pltpu.sync_copy(bias_hbm, bias_v)

    lane = lax.iota(jnp.int32, 16)
    zero = jnp.zeros((16,), jnp.int32)
    one = jnp.full((16,), 1, jnp.int32)
    e_vecs = [jnp.full((16,), e, jnp.int32) for e in range(_N_EXPERTS)]
    bias_b = [plsc.load_gather(bias_v, [e_vecs[e]]) for e in range(_N_EXPERTS)]

    def _comb(A, B):
        (a1m, a1i, a2m, a2i), (b1m, b1i, b2m, b2i) = A, B
        c1 = b1m > a1m
        c2 = b1m > a2m
        c3 = b2m > a1m
        n1m = jnp.where(c1, b1m, a1m)
        n1i = jnp.where(c1, b1i, a1i)
        n2m = jnp.where(c1, jnp.where(c3, b2m, a1m), jnp.where(c2, b1m, a2m))
        n2i = jnp.where(c1, jnp.where(c3, b2i, a1i), jnp.where(c2, b1i, a2i))
        return (n1m, n1i, n2m, n2i)

    def group(t, carry):
        base = t * _LANES
        tok = base + lane
        s_sel = []
        for e in range(_N_EXPERTS):
            z = logit_v[e, pl.ds(base, _LANES)]
            s_sel.append(1.0 / (1.0 + jnp.exp(-z)) + bias_b[e])
        nodes = []
        for e in range(0, _N_EXPERTS, 2):
            c = s_sel[e + 1] > s_sel[e]
            nodes.append((
                jnp.where(c, s_sel[e + 1], s_sel[e]),
                jnp.where(c, e_vecs[e + 1], e_vecs[e]),
                jnp.where(c, s_sel[e], s_sel[e + 1]),
                jnp.where(c, e_vecs[e], e_vecs[e + 1]),
            ))
        while len(nodes) > 1:
            nodes = [_comb(nodes[i], nodes[i + 1])
                     for i in range(0, len(nodes), 2)]
        m1, i1, m2, i2 = nodes[0]
        w1 = m1 - plsc.load_gather(bias_v, [i1])
        w2 = m2 - plsc.load_gather(bias_v, [i2])
        denom = jnp.maximum(w1 + w2, 1e-12)
        scale = 1.0 / denom
        w_v[0, pl.ds(base, _LANES)] = w1 * scale
        w_v[1, pl.ds(base, _LANES)] = w2 * scale
        i_v[0, pl.ds(base, _LANES)] = i1
        i_v[1, pl.ds(base, _LANES)] = i2
        return carry

    lax.fori_loop(0, n_tok // _LANES, group, 0, unroll=2)

    pltpu.sync_copy(w_v, w_out_hbm.at[:, pl.ds(tok0, n_tok)])
    pltpu.sync_copy(i_v, idx_out_hbm.at[:, pl.ds(tok0, n_tok)])


def _sc_router(logits_t, bias):
    tokens = logits_t.shape[1]
    n_tok = tokens // _NW
    mesh = plsc.VectorSubcoreMesh(core_axis_name="c", subcore_axis_name="s")
    run = pl.kernel(
        _sc_router_body,
        out_type=[
            jax.ShapeDtypeStruct((_TOPK, tokens), jnp.float32),
            jax.ShapeDtypeStruct((_TOPK, tokens), jnp.int32),
        ],
        mesh=mesh,
        scratch_types=[
            pltpu.VMEM((_N_EXPERTS, n_tok), jnp.float32),
            pltpu.VMEM((_N_EXPERTS,), jnp.float32),
            pltpu.VMEM((_TOPK, n_tok), jnp.float32),
            pltpu.VMEM((_TOPK, n_tok), jnp.int32),
        ],
        compiler_params=pltpu.CompilerParams(needs_layout_passes=False),
    )
    return run(logits_t, bias)


def _repack_body(wt_ref, it_ref, w_ref, i_ref):
    w_ref[...] = wt_ref[...].T
    i_ref[...] = it_ref[...].T


def _tc_repack(w_t, i_t, tokens):
    return pl.pallas_call(
        _repack_body,
        in_specs=[
            pl.BlockSpec((_TOPK, tokens), lambda: (0, 0)),
            pl.BlockSpec((_TOPK, tokens), lambda: (0, 0)),
        ],
        out_specs=[
            pl.BlockSpec((tokens, _TOPK), lambda: (0, 0)),
            pl.BlockSpec((tokens, _TOPK), lambda: (0, 0)),
        ],
        out_shape=[
            jax.ShapeDtypeStruct((tokens, _TOPK), jnp.float32),
            jax.ShapeDtypeStruct((tokens, _TOPK), jnp.int32),
        ],
    )(w_t, i_t)


@jax.jit
def kernel(x, W, bias):
    tokens = x.shape[0]
    logits_t = _tc_logits_t(x, W)
    w_t, i_t = _sc_router(logits_t, bias)
    return _tc_repack(w_t, i_t, tokens)


# speedup vs baseline: 1.4825x; 1.3167x over previous
import functools

import jax
import jax.numpy as jnp
from jax import lax
from jax.experimental import pallas as pl
from jax.experimental.pallas import tpu as pltpu
from jax.experimental.pallas import tpu_sc as plsc

_N_EXPERTS = 16
_TOPK = 2
_BT = 1024

_NC = 2
_NS = 16
_NW = _NC * _NS
_LANES = 16


def _matmul_body(x_ref, w_ref, out_ref):
    out_ref[...] = lax.dot_general(
        w_ref[...], x_ref[...], (((1,), (1,)), ((), ())),
        preferred_element_type=jnp.float32)


def _tc_logits_t(x, W):
    tokens, dim = x.shape
    n_experts = W.shape[0]
    return pl.pallas_call(
        _matmul_body,
        grid=(tokens // _BT,),
        in_specs=[
            pl.BlockSpec((_BT, dim), lambda i: (i, 0)),
            pl.BlockSpec((n_experts, dim), lambda i: (0, 0)),
        ],
        out_specs=pl.BlockSpec((n_experts, _BT), lambda i: (0, i)),
        out_shape=jax.ShapeDtypeStruct((n_experts, tokens), jnp.float32),
    )(x, W)


def _sc_router_body(logits_hbm, bias_hbm, w_out_hbm, idx_out_hbm,
                    logit_v, bias_v, w_v, i_v):
    n_tok = logit_v.shape[1]
    wid = lax.axis_index("s") * _NC + lax.axis_index("c")
    tok0 = wid * n_tok
    pltpu.sync_copy(logits_hbm.at[:, pl.ds(tok0, n_tok)], logit_v)
    pltpu.sync_copy(bias_hbm, bias_v)

    lane = lax.iota(jnp.int32, 16)
    zero = jnp.zeros((16,), jnp.int32)
    one = jnp.full((16,), 1, jnp.int32)
    e_vecs = [jnp.full((16,), e, jnp.int32) for e in range(_N_EXPERTS)]
    bias_b = [plsc.load_gather(bias_v, [e_vecs[e]]) for e in range(_N_EXPERTS)]

    def _comb(A, B):
        (a1m, a1i, a2m, a2i), (b1m, b1i, b2m, b2i) = A, B
        c1 = b1m > a1m
        c2 = b1m > a2m
        c3 = b2m > a1m
        n1m = jnp.where(c1, b1m, a1m)
        n1i = jnp.where(c1, b1i, a1i)
        n2m = jnp.where(c1, jnp.where(c3, b2m, a1m), jnp.where(c2, b1m, a2m))
        n2i = jnp.where(c1, jnp.where(c3, b2i, a1i), jnp.where(c2, b1i, a2i))
        return (n1m, n1i, n2m, n2i)

    def group(t, carry):
        base = t * _LANES
        tok = base + lane
        s_sel = []
        for e in range(_N_EXPERTS):
            z = logit_v[e, pl.ds(base, _LANES)]
            s_sel.append(1.0 / (1.0 + jnp.exp(-z)) + bias_b[e])
        nodes = []
        for e in range(0, _N_EXPERTS, 2):
            c = s_sel[e + 1] > s_sel[e]
            nodes.append((
                jnp.where(c, s_sel[e + 1], s_sel[e]),
                jnp.where(c, e_vecs[e + 1], e_vecs[e]),
                jnp.where(c, s_sel[e], s_sel[e + 1]),
                jnp.where(c, e_vecs[e], e_vecs[e + 1]),
            ))
        while len(nodes) > 1:
            nodes = [_comb(nodes[i], nodes[i + 1])
                     for i in range(0, len(nodes), 2)]
        m1, i1, m2, i2 = nodes[0]
        w1 = m1 - plsc.load_gather(bias_v, [i1])
        w2 = m2 - plsc.load_gather(bias_v, [i2])
        denom = jnp.maximum(w1 + w2, 1e-12)
        scale = 1.0 / denom
        w_v[0, pl.ds(base, _LANES)] = w1 * scale
        w_v[1, pl.ds(base, _LANES)] = w2 * scale
        i_v[0, pl.ds(base, _LANES)] = i1
        i_v[1, pl.ds(base, _LANES)] = i2
        return carry

    lax.fori_loop(0, n_tok // _LANES, group, 0, unroll=2)

    pltpu.sync_copy(w_v, w_out_hbm.at[:, pl.ds(tok0, n_tok)])
    pltpu.sync_copy(i_v, idx_out_hbm.at[:, pl.ds(tok0, n_tok)])


def _sc_router(logits_t, bias):
    tokens = logits_t.shape[1]
    n_tok = tokens // _NW
    mesh = plsc.VectorSubcoreMesh(core_axis_name="c", subcore_axis_name="s")
    run = pl.kernel(
        _sc_router_body,
        out_type=[
            jax.ShapeDtypeStruct((_TOPK, tokens), jnp.float32),
            jax.ShapeDtypeStruct((_TOPK, tokens), jnp.int32),
        ],
        mesh=mesh,
        scratch_types=[
            pltpu.VMEM((_N_EXPERTS, n_tok), jnp.float32),
            pltpu.VMEM((_N_EXPERTS,), jnp.float32),
            pltpu.VMEM((_TOPK, n_tok), jnp.float32),
            pltpu.VMEM((_TOPK, n_tok), jnp.int32),
        ],
        compiler_params=pltpu.CompilerParams(needs_layout_passes=False),
    )
    return run(logits_t, bias)


@jax.jit
def kernel(x, W, bias):
    logits_t = _tc_logits_t(x, W)
    w_t, i_t = _sc_router(logits_t, bias)
    return w_t.T, i_t.T
